# Initial kernel scaffold; baseline (speedup 1.0000x reference)
import jax
import jax.numpy as jnp
from jax.experimental import pallas as pl


def kernel(abs_pos, vel_hist, senders, receivers):
    raise NotImplementedError("write your pallas kernel here")


# speedup vs baseline: 152.1866x; 152.1866x over previous
import jax
import jax.numpy as jnp
import numpy as np
from jax import lax
from jax.experimental import pallas as pl
from jax.experimental.pallas import tpu as pltpu
from jax.experimental.pallas import tpu_sc as plsc

_N = 100000
_E = 3200000
_DIM = 3
_ROW = 128
_NTILES = 32
_BA = 4
_NSTEPS = 196
_ROWS_PAD = _NTILES * _BA * _NSTEPS
_E_PAD = _ROWS_PAD * _ROW
_NSLICE = 6400
_N_PAD = 16 * _NSLICE

_SIGMA = float(3.0 / (359.0 * np.pi))
_ETA = np.float32(np.float32(2.0) * np.float32(0.01) * np.float32(0.01)
                  / (np.float32(0.01) + np.float32(0.01) + np.float32(1e-8)))
_P_REF = 100.0

_mesh = plsc.VectorSubcoreMesh(core_axis_name="c", subcore_axis_name="s",
                               num_cores=2, num_subcores=16)
_sc_params = pltpu.CompilerParams(needs_layout_passes=False,
                                  use_tc_tiling_on_sc=False)


def _iota16():
    return lax.iota(jnp.int32, 16)


def _splat_i32(v):
    return jnp.full((16,), v, dtype=jnp.int32)


def _rsqrt(sq):
    i = plsc.bitcast(sq, jnp.int32)
    y = plsc.bitcast(jnp.int32(0x5F3759DF) - (i >> 1), jnp.float32)
    for _ in range(3):
        y = y * (1.5 - 0.5 * sq * y * y)
    return y


def _zero_1d(ref, n):
    def _z(i, _):
        ref[pl.ds(16 * i, 16)] = jnp.zeros((16,), jnp.float32)
        return 0
    lax.fori_loop(0, n // 16, _z, 0)


def _issue(tab_hbm, s2d_hbm, r2d_hbm, wid, k, buf):
    sidx, ridx, rows_i, rows_j, sem_i, sem_j = buf
    row0 = (wid + _NTILES * k) * _BA
    pltpu.sync_copy(s2d_hbm.at[pl.ds(row0, _BA)], sidx)
    pltpu.sync_copy(r2d_hbm.at[pl.ds(row0, _BA)], ridx)
    for r in range(_BA):
        pltpu.async_copy(tab_hbm.at[sidx.at[r]], rows_i.at[r], sem_i)
        pltpu.async_copy(tab_hbm.at[ridx.at[r]], rows_j.at[r], sem_j)


def _drain(tab_hbm, buf):
    sidx, ridx, rows_i, rows_j, sem_i, sem_j = buf
    for r in range(_BA):
        pltpu.make_async_copy(tab_hbm.at[sidx.at[r]], rows_i.at[r], sem_i).wait()
        pltpu.make_async_copy(tab_hbm.at[ridx.at[r]], rows_j.at[r], sem_j).wait()


def _rho_body(pos_hbm, snd_hbm, rcv_hbm, out_hbm,
              sidx0, ridx0, ri0, rj0, sidx1, ridx1, ri1, rj1,
              wbuf, zbuf, shared_rho,
              semi0, semj0, semi1, semj1):
    c = lax.axis_index("c")
    s = lax.axis_index("s")
    wid = c * 16 + s
    bufs = ((sidx0, ridx0, ri0, rj0, semi0, semj0),
            (sidx1, ridx1, ri1, rj1, semi1, semj1))

    _zero_1d(zbuf, _NSLICE)
    pltpu.sync_copy(zbuf, shared_rho.at[pl.ds(s * _NSLICE, _NSLICE)])
    plsc.subcore_barrier()

    def _compute(buf):
        sidx, ridx, rows_i, rows_j, _, _ = buf
        load = plsc.load_gather
        for r in range(_BA):
            ri = rows_i.at[r]
            rj = rows_j.at[r]
            for g in range(_ROW // 16):
                e = _splat_i32(16 * g) + _iota16()
                dx = load(ri, [e, _splat_i32(0)]) - load(rj, [e, _splat_i32(0)])
                dy = load(ri, [e, _splat_i32(1)]) - load(rj, [e, _splat_i32(1)])
                dz = load(ri, [e, _splat_i32(2)]) - load(rj, [e, _splat_i32(2)])
                sq = dx * dx + dy * dy + dz * dz
                d = sq * _rsqrt(sq)
                q1 = jnp.maximum(1.0 - d, 0.0)
                q2 = jnp.maximum(2.0 - d, 0.0)
                q3 = jnp.maximum(3.0 - d, 0.0)
                a1 = q1 * q1
                a2 = q2 * q2
                a3 = q3 * q3
                w = _SIGMA * ((a3 * a3) * q3 - 6.0 * (a2 * a2) * q2
                              + 15.0 * (a1 * a1) * q1)
                wbuf[r, pl.ds(16 * g, 16)] = w
        for r in range(_BA):
            pltpu.sync_copy(wbuf.at[r], shared_rho.at[sidx.at[r]], add=True)

    _issue(pos_hbm, snd_hbm, rcv_hbm, wid, 0, bufs[0])
    _issue(pos_hbm, snd_hbm, rcv_hbm, wid, 1, bufs[1])

    def _outer(k2, _):
        _drain(pos_hbm, bufs[0])
        _compute(bufs[0])

        @pl.when(k2 < _NSTEPS // 2 - 1)
        def _():
            _issue(pos_hbm, snd_hbm, rcv_hbm, wid, 2 * k2 + 2, bufs[0])

        _drain(pos_hbm, bufs[1])
        _compute(bufs[1])

        @pl.when(k2 < _NSTEPS // 2 - 1)
        def _():
            _issue(pos_hbm, snd_hbm, rcv_hbm, wid, 2 * k2 + 3, bufs[1])
        return 0

    lax.fori_loop(0, _NSTEPS // 2, _outer, 0)
    plsc.subcore_barrier()

    sl = pl.ds(s * _NSLICE, _NSLICE)
    pltpu.sync_copy(shared_rho.at[sl], zbuf)
    pltpu.sync_copy(zbuf, out_hbm.at[pl.ds(c * _N_PAD + s * _NSLICE, _NSLICE)])


def _edge_scratch():
    return [
        pltpu.VMEM((_BA, _ROW), jnp.int32),
        pltpu.VMEM((_BA, _ROW), jnp.int32),
        pltpu.VMEM((_BA, _ROW, 8), jnp.float32),
        pltpu.VMEM((_BA, _ROW, 8), jnp.float32),
    ]


_rho_call = pl.kernel(
    _rho_body,
    out_type=jax.ShapeDtypeStruct((2 * _N_PAD,), jnp.float32),
    mesh=_mesh,
    compiler_params=_sc_params,
    scratch_types=(
        _edge_scratch() + _edge_scratch() + [
            pltpu.VMEM((_BA, _ROW), jnp.float32),
            pltpu.VMEM((_NSLICE,), jnp.float32),
            pltpu.VMEM_SHARED((_N_PAD,), jnp.float32),
            pltpu.SemaphoreType.DMA,
            pltpu.SemaphoreType.DMA,
            pltpu.SemaphoreType.DMA,
            pltpu.SemaphoreType.DMA,
        ]
    ),
)


_BLK = 4096


def _pack_body(part_ref, pos_ref, vel_ref, tab_ref, rho_ref, p_ref):
    rho = part_ref[0, :] + part_ref[1, :]
    tab_ref[...] = jnp.concatenate(
        [pos_ref[...], vel_ref[...], rho[:, None],
         jnp.zeros((_BLK, 1), jnp.float32)], axis=1)
    rho_ref[...] = rho
    p_ref[...] = _P_REF * (rho - 1.0)


_pack_call = pl.pallas_call(
    _pack_body,
    grid=(_N_PAD // _BLK,),
    in_specs=[
        pl.BlockSpec((2, _BLK), lambda i: (0, i)),
        pl.BlockSpec((_BLK, _DIM), lambda i: (i, 0)),
        pl.BlockSpec((_BLK, _DIM), lambda i: (i, 0)),
    ],
    out_specs=[
        pl.BlockSpec((_BLK, 8), lambda i: (i, 0)),
        pl.BlockSpec((_BLK,), lambda i: (i,)),
        pl.BlockSpec((_BLK,), lambda i: (i,)),
    ],
    out_shape=[
        jax.ShapeDtypeStruct((_N_PAD, 8), jnp.float32),
        jax.ShapeDtypeStruct((_N_PAD,), jnp.float32),
        jax.ShapeDtypeStruct((_N_PAD,), jnp.float32),
    ],
)


def _force_body(tab_hbm, snd_hbm, rcv_hbm, out_hbm,
                sidx0, ridx0, ri0, rj0, sidx1, ridx1, ri1, rj1,
                vx, vy, vz, zbuf, shx, shy, shz,
                semi0, semj0, semi1, semj1):
    c = lax.axis_index("c")
    s = lax.axis_index("s")
    wid = c * 16 + s
    bufs = ((sidx0, ridx0, ri0, rj0, semi0, semj0),
            (sidx1, ridx1, ri1, rj1, semi1, semj1))

    _zero_1d(zbuf, _NSLICE)
    sl = pl.ds(s * _NSLICE, _NSLICE)
    pltpu.sync_copy(zbuf, shx.at[sl])
    pltpu.sync_copy(zbuf, shy.at[sl])
    pltpu.sync_copy(zbuf, shz.at[sl])
    plsc.subcore_barrier()

    def _compute(buf):
        sidx, ridx, rows_i, rows_j, _, _ = buf
        load = plsc.load_gather
        for r in range(_BA):
            ri = rows_i.at[r]
            rj = rows_j.at[r]
            for g in range(_ROW // 16):
                e = _splat_i32(16 * g) + _iota16()
                dx = load(ri, [e, _splat_i32(0)]) - load(rj, [e, _splat_i32(0)])
                dy = load(ri, [e, _splat_i32(1)]) - load(rj, [e, _splat_i32(1)])
                dz = load(ri, [e, _splat_i32(2)]) - load(rj, [e, _splat_i32(2)])
                ux = load(ri, [e, _splat_i32(3)]) - load(rj, [e, _splat_i32(3)])
                uy = load(ri, [e, _splat_i32(4)]) - load(rj, [e, _splat_i32(4)])
                uz = load(ri, [e, _splat_i32(5)]) - load(rj, [e, _splat_i32(5)])
                rho_i = load(ri, [e, _splat_i32(6)])
                rho_j = load(rj, [e, _splat_i32(6)])
                sq = dx * dx + dy * dy + dz * dz
                d = sq * _rsqrt(sq)
                q1 = jnp.maximum(1.0 - d, 0.0)
                q2 = jnp.maximum(2.0 - d, 0.0)
                q3 = jnp.maximum(3.0 - d, 0.0)
                a1 = q1 * q1
                a2 = q2 * q2
                a3 = q3 * q3
                gw = _SIGMA * (-5.0 * (a3 * a3) + 30.0 * (a2 * a2)
                               - 75.0 * (a1 * a1))
                inv_i = 1.0 / rho_i
                inv_j = 1.0 / rho_j
                wv = inv_i * inv_i + inv_j * inv_j
                c_ = wv * gw / (d + 1e-8)
                p_i = _P_REF * (rho_i - 1.0)
                p_j = _P_REF * (rho_j - 1.0)
                p_ij = (rho_j * p_i + rho_i * p_j) / (rho_i + rho_j)
                g16 = pl.ds(16 * g, 16)
                vx[r, g16] = c_ * (_ETA * ux - p_ij * dx)
                vy[r, g16] = c_ * (_ETA * uy - p_ij * dy)
                vz[r, g16] = c_ * (_ETA * uz - p_ij * dz)
        for r in range(_BA):
            si = sidx.at[r]
            pltpu.sync_copy(vx.at[r], shx.at[si], add=True)
            pltpu.sync_copy(vy.at[r], shy.at[si], add=True)
            pltpu.sync_copy(vz.at[r], shz.at[si], add=True)

    _issue(tab_hbm, snd_hbm, rcv_hbm, wid, 0, bufs[0])
    _issue(tab_hbm, snd_hbm, rcv_hbm, wid, 1, bufs[1])

    def _outer(k2, _):
        _drain(tab_hbm, bufs[0])
        _compute(bufs[0])

        @pl.when(k2 < _NSTEPS // 2 - 1)
        def _():
            _issue(tab_hbm, snd_hbm, rcv_hbm, wid, 2 * k2 + 2, bufs[0])

        _drain(tab_hbm, bufs[1])
        _compute(bufs[1])

        @pl.when(k2 < _NSTEPS // 2 - 1)
        def _():
            _issue(tab_hbm, snd_hbm, rcv_hbm, wid, 2 * k2 + 3, bufs[1])
        return 0

    lax.fori_loop(0, _NSTEPS // 2, _outer, 0)
    plsc.subcore_barrier()

    for comp, sh in enumerate((shx, shy, shz)):
        pltpu.sync_copy(sh.at[sl], zbuf)
        pltpu.sync_copy(
            zbuf,
            out_hbm.at[pl.ds((c * _DIM + comp) * _N_PAD + s * _NSLICE, _NSLICE)])


_force_call = pl.kernel(
    _force_body,
    out_type=jax.ShapeDtypeStruct((2 * _DIM * _N_PAD,), jnp.float32),
    mesh=_mesh,
    compiler_params=_sc_params,
    scratch_types=(
        _edge_scratch() + _edge_scratch() + [
            pltpu.VMEM((_BA, _ROW), jnp.float32),
            pltpu.VMEM((_BA, _ROW), jnp.float32),
            pltpu.VMEM((_BA, _ROW), jnp.float32),
            pltpu.VMEM((_NSLICE,), jnp.float32),
            pltpu.VMEM_SHARED((_N_PAD,), jnp.float32),
            pltpu.VMEM_SHARED((_N_PAD,), jnp.float32),
            pltpu.VMEM_SHARED((_N_PAD,), jnp.float32),
            pltpu.SemaphoreType.DMA,
            pltpu.SemaphoreType.DMA,
            pltpu.SemaphoreType.DMA,
            pltpu.SemaphoreType.DMA,
        ]
    ),
)


def _du_body(part_ref, out_ref):
    out_ref[...] = part_ref[0] + part_ref[1]


_du_call = pl.pallas_call(
    _du_body,
    grid=(_N_PAD // _BLK,),
    in_specs=[pl.BlockSpec((2, _DIM, _BLK), lambda i: (0, 0, i))],
    out_specs=pl.BlockSpec((_DIM, _BLK), lambda i: (0, i)),
    out_shape=jax.ShapeDtypeStruct((_DIM, _N_PAD), jnp.float32),
)


def kernel(abs_pos, vel_hist, senders, receivers):
    pad = jnp.full((_E_PAD - _E,), _N, jnp.int32)
    s2d = jnp.concatenate([senders, pad]).reshape(_ROWS_PAD, _ROW)
    r2d = jnp.concatenate([receivers, pad]).reshape(_ROWS_PAD, _ROW)
    pos8 = jnp.pad(abs_pos, ((0, _N_PAD - _N), (0, 5)))
    partial_rho = _rho_call(pos8, s2d, r2d).reshape(2, _N_PAD)
    table, rho, p = _pack_call(partial_rho, abs_pos, vel_hist)
    partial_du = _force_call(table, s2d, r2d)
    du3 = _du_call(partial_du.reshape(2, _DIM, _N_PAD))
    dudt = du3[:, :_N].T
    dvdt = jnp.zeros((_N, _DIM), jnp.float32)
    return (dudt, dvdt, rho[:_N], p[:_N])


# speedup vs baseline: 166.8157x; 1.0961x over previous
import jax
import jax.numpy as jnp
import numpy as np
from jax import lax
from jax.experimental import pallas as pl
from jax.experimental.pallas import tpu as pltpu
from jax.experimental.pallas import tpu_sc as plsc

_N = 100000
_E = 3200000
_DIM = 3
_ROW = 128
_NTILES = 32
_BA = 4
_NSTEPS = 196
_BA_A = 8
_NSTEPS_A = 98
_ROWS_PAD = _NTILES * _BA * _NSTEPS
_E_PAD = _ROWS_PAD * _ROW
_NSLICE = 6400
_N_PAD = 16 * _NSLICE

_SIGMA = float(3.0 / (359.0 * np.pi))
_ETA = np.float32(np.float32(2.0) * np.float32(0.01) * np.float32(0.01)
                  / (np.float32(0.01) + np.float32(0.01) + np.float32(1e-8)))
_P_REF = 100.0

_mesh = plsc.VectorSubcoreMesh(core_axis_name="c", subcore_axis_name="s",
                               num_cores=2, num_subcores=16)
_sc_params = pltpu.CompilerParams(needs_layout_passes=False,
                                  use_tc_tiling_on_sc=False)


def _iota16():
    return lax.iota(jnp.int32, 16)


def _splat_i32(v):
    return jnp.full((16,), v, dtype=jnp.int32)


def _rsqrt(sq):
    i = plsc.bitcast(sq, jnp.int32)
    y = plsc.bitcast(jnp.int32(0x5F3759DF) - (i >> 1), jnp.float32)
    for _ in range(3):
        y = y * (1.5 - 0.5 * sq * y * y)
    return y


def _copy_idx(src, dst, ba):
    for r in range(ba):
        for g in range(_ROW // 16):
            sl = pl.ds(16 * g, 16)
            dst[r, sl] = src[r, sl]


def _zero_1d(ref, n):
    def _z(i, _):
        ref[pl.ds(16 * i, 16)] = jnp.zeros((16,), jnp.float32)
        return 0
    lax.fori_loop(0, n // 16, _z, 0)


def _issue(tab_hbm, s2d_hbm, r2d_hbm, wid, k, buf, ba):
    sidx, ridx, rows_i, rows_j, sem_i, sem_j = buf
    row0 = (wid + _NTILES * k) * ba
    pltpu.sync_copy(s2d_hbm.at[pl.ds(row0, ba)], sidx)
    pltpu.sync_copy(r2d_hbm.at[pl.ds(row0, ba)], ridx)
    for r in range(ba):
        pltpu.async_copy(tab_hbm.at[sidx.at[r]], rows_i.at[r], sem_i)
        pltpu.async_copy(tab_hbm.at[ridx.at[r]], rows_j.at[r], sem_j)


def _drain(tab_hbm, buf, ba):
    sidx, ridx, rows_i, rows_j, sem_i, sem_j = buf
    for r in range(ba):
        pltpu.make_async_copy(tab_hbm.at[sidx.at[r]], rows_i.at[r], sem_i).wait()
        pltpu.make_async_copy(tab_hbm.at[ridx.at[r]], rows_j.at[r], sem_j).wait()


def _rho_body(pos_hbm, snd_hbm, rcv_hbm, out_hbm,
              sidx0, ridx0, ri0, rj0, sidx1, ridx1, ri1, rj1,
              wbuf0, wbuf1, ssx0, ssx1, zbuf, shared_rho,
              semi0, semj0, semi1, semj1, semsc0, semsc1):
    c = lax.axis_index("c")
    s = lax.axis_index("s")
    wid = c * 16 + s
    bufs = ((sidx0, ridx0, ri0, rj0, semi0, semj0),
            (sidx1, ridx1, ri1, rj1, semi1, semj1))
    wbufs = (wbuf0, wbuf1)
    ssxs = (ssx0, ssx1)
    semscs = (semsc0, semsc1)

    _zero_1d(zbuf, _NSLICE)
    pltpu.sync_copy(zbuf, shared_rho.at[pl.ds(s * _NSLICE, _NSLICE)])
    plsc.subcore_barrier()

    def _compute(b, k):
        sidx, ridx, rows_i, rows_j, _, _ = bufs[b]
        wb, ssx, semsc = wbufs[b], ssxs[b], semscs[b]

        @pl.when(k >= 2)
        def _():
            for r in range(_BA_A):
                pltpu.make_async_copy(
                    wb.at[r], shared_rho.at[ssx.at[r]], semsc).wait()

        load = plsc.load_gather
        for r in range(_BA_A):
            ri = rows_i.at[r]
            rj = rows_j.at[r]
            for g in range(_ROW // 16):
                e = _splat_i32(16 * g) + _iota16()
                dx = load(ri, [e, _splat_i32(0)]) - load(rj, [e, _splat_i32(0)])
                dy = load(ri, [e, _splat_i32(1)]) - load(rj, [e, _splat_i32(1)])
                dz = load(ri, [e, _splat_i32(2)]) - load(rj, [e, _splat_i32(2)])
                sq = dx * dx + dy * dy + dz * dz
                d = sq * _rsqrt(sq)
                q1 = jnp.maximum(1.0 - d, 0.0)
                q2 = jnp.maximum(2.0 - d, 0.0)
                q3 = jnp.maximum(3.0 - d, 0.0)
                a1 = q1 * q1
                a2 = q2 * q2
                a3 = q3 * q3
                w = _SIGMA * ((a3 * a3) * q3 - 6.0 * (a2 * a2) * q2
                              + 15.0 * (a1 * a1) * q1)
                wb[r, pl.ds(16 * g, 16)] = w
        _copy_idx(sidx, ssx, _BA_A)
        for r in range(_BA_A):
            pltpu.async_copy(wb.at[r], shared_rho.at[ssx.at[r]], semsc, add=True)

    _issue(pos_hbm, snd_hbm, rcv_hbm, wid, 0, bufs[0], _BA_A)
    _issue(pos_hbm, snd_hbm, rcv_hbm, wid, 1, bufs[1], _BA_A)

    def _outer(k2, _):
        _drain(pos_hbm, bufs[0], _BA_A)
        _compute(0, 2 * k2)

        @pl.when(k2 < _NSTEPS_A // 2 - 1)
        def _():
            _issue(pos_hbm, snd_hbm, rcv_hbm, wid, 2 * k2 + 2, bufs[0], _BA_A)

        _drain(pos_hbm, bufs[1], _BA_A)
        _compute(1, 2 * k2 + 1)

        @pl.when(k2 < _NSTEPS_A // 2 - 1)
        def _():
            _issue(pos_hbm, snd_hbm, rcv_hbm, wid, 2 * k2 + 3, bufs[1], _BA_A)
        return 0

    lax.fori_loop(0, _NSTEPS_A // 2, _outer, 0)
    for b in (0, 1):
        for r in range(_BA_A):
            pltpu.make_async_copy(
                wbufs[b].at[r], shared_rho.at[ssxs[b].at[r]], semscs[b]).wait()
    plsc.subcore_barrier()

    sl = pl.ds(s * _NSLICE, _NSLICE)
    pltpu.sync_copy(shared_rho.at[sl], zbuf)
    pltpu.sync_copy(zbuf, out_hbm.at[pl.ds(c * _N_PAD + s * _NSLICE, _NSLICE)])


def _edge_scratch(ba):
    return [
        pltpu.VMEM((ba, _ROW), jnp.int32),
        pltpu.VMEM((ba, _ROW), jnp.int32),
        pltpu.VMEM((ba, _ROW, 8), jnp.float32),
        pltpu.VMEM((ba, _ROW, 8), jnp.float32),
    ]


_rho_call = pl.kernel(
    _rho_body,
    out_type=jax.ShapeDtypeStruct((2 * _N_PAD,), jnp.float32),
    mesh=_mesh,
    compiler_params=_sc_params,
    scratch_types=(
        _edge_scratch(_BA_A) + _edge_scratch(_BA_A) + [
            pltpu.VMEM((_BA_A, _ROW), jnp.float32),
            pltpu.VMEM((_BA_A, _ROW), jnp.float32),
            pltpu.VMEM((_BA_A, _ROW), jnp.int32),
            pltpu.VMEM((_BA_A, _ROW), jnp.int32),
            pltpu.VMEM((_NSLICE,), jnp.float32),
            pltpu.VMEM_SHARED((_N_PAD,), jnp.float32),
        ] + [pltpu.SemaphoreType.DMA] * 6
    ),
)


_BLK = 4096


def _pack_body(part_ref, pos_ref, vel_ref, tab_ref, rho_ref, p_ref):
    rho = part_ref[0, :] + part_ref[1, :]
    tab_ref[...] = jnp.concatenate(
        [pos_ref[...], vel_ref[...], rho[:, None],
         jnp.zeros((_BLK, 1), jnp.float32)], axis=1)
    rho_ref[...] = rho
    p_ref[...] = _P_REF * (rho - 1.0)


_pack_call = pl.pallas_call(
    _pack_body,
    grid=(_N_PAD // _BLK,),
    in_specs=[
        pl.BlockSpec((2, _BLK), lambda i: (0, i)),
        pl.BlockSpec((_BLK, _DIM), lambda i: (i, 0)),
        pl.BlockSpec((_BLK, _DIM), lambda i: (i, 0)),
    ],
    out_specs=[
        pl.BlockSpec((_BLK, 8), lambda i: (i, 0)),
        pl.BlockSpec((_BLK,), lambda i: (i,)),
        pl.BlockSpec((_BLK,), lambda i: (i,)),
    ],
    out_shape=[
        jax.ShapeDtypeStruct((_N_PAD, 8), jnp.float32),
        jax.ShapeDtypeStruct((_N_PAD,), jnp.float32),
        jax.ShapeDtypeStruct((_N_PAD,), jnp.float32),
    ],
)


def _force_body(tab_hbm, snd_hbm, rcv_hbm, out_hbm,
                sidx0, ridx0, ri0, rj0, sidx1, ridx1, ri1, rj1,
                vx0, vy0, vz0, vx1, vy1, vz1, ssx0, ssx1,
                zbuf, stage, shared_tab, shx, shy, shz,
                semi0, semj0, semi1, semj1, semsc0, semsc1):
    c = lax.axis_index("c")
    s = lax.axis_index("s")
    wid = c * 16 + s
    bufs = ((sidx0, ridx0, ri0, rj0, semi0, semj0),
            (sidx1, ridx1, ri1, rj1, semi1, semj1))
    vals = ((vx0, vy0, vz0), (vx1, vy1, vz1))
    ssxs = (ssx0, ssx1)
    semscs = (semsc0, semsc1)

    _zero_1d(zbuf, _NSLICE)
    sl = pl.ds(s * _NSLICE, _NSLICE)
    pltpu.sync_copy(zbuf, shx.at[sl])
    pltpu.sync_copy(zbuf, shy.at[sl])
    pltpu.sync_copy(zbuf, shz.at[sl])
    for i in range(_NSLICE // 1600):
        tsl = pl.ds(s * _NSLICE + i * 1600, 1600)
        pltpu.sync_copy(tab_hbm.at[tsl], stage)
        pltpu.sync_copy(stage, shared_tab.at[tsl])
    plsc.subcore_barrier()

    def _drain_sc(b):
        vx, vy, vz = vals[b]
        ssx, semsc = ssxs[b], semscs[b]
        for r in range(_BA):
            si = ssx.at[r]
            pltpu.make_async_copy(vx.at[r], shx.at[si], semsc).wait()
            pltpu.make_async_copy(vy.at[r], shy.at[si], semsc).wait()
            pltpu.make_async_copy(vz.at[r], shz.at[si], semsc).wait()

    def _compute(b, k):
        sidx, ridx, rows_i, rows_j, _, _ = bufs[b]
        vx, vy, vz = vals[b]
        ssx, semsc = ssxs[b], semscs[b]

        @pl.when(k >= 2)
        def _():
            _drain_sc(b)

        load = plsc.load_gather
        for r in range(_BA):
            ri = rows_i.at[r]
            rj = rows_j.at[r]
            for g in range(_ROW // 16):
                e = _splat_i32(16 * g) + _iota16()
                dx = load(ri, [e, _splat_i32(0)]) - load(rj, [e, _splat_i32(0)])
                dy = load(ri, [e, _splat_i32(1)]) - load(rj, [e, _splat_i32(1)])
                dz = load(ri, [e, _splat_i32(2)]) - load(rj, [e, _splat_i32(2)])
                ux = load(ri, [e, _splat_i32(3)]) - load(rj, [e, _splat_i32(3)])
                uy = load(ri, [e, _splat_i32(4)]) - load(rj, [e, _splat_i32(4)])
                uz = load(ri, [e, _splat_i32(5)]) - load(rj, [e, _splat_i32(5)])
                rho_i = load(ri, [e, _splat_i32(6)])
                rho_j = load(rj, [e, _splat_i32(6)])
                sq = dx * dx + dy * dy + dz * dz
                d = sq * _rsqrt(sq)
                q1 = jnp.maximum(1.0 - d, 0.0)
                q2 = jnp.maximum(2.0 - d, 0.0)
                q3 = jnp.maximum(3.0 - d, 0.0)
                a1 = q1 * q1
                a2 = q2 * q2
                a3 = q3 * q3
                gw = _SIGMA * (-5.0 * (a3 * a3) + 30.0 * (a2 * a2)
                               - 75.0 * (a1 * a1))
                inv_i = 1.0 / rho_i
                inv_j = 1.0 / rho_j
                wv = inv_i * inv_i + inv_j * inv_j
                c_ = wv * gw / (d + 1e-8)
                p_i = _P_REF * (rho_i - 1.0)
                p_j = _P_REF * (rho_j - 1.0)
                p_ij = (rho_j * p_i + rho_i * p_j) / (rho_i + rho_j)
                g16 = pl.ds(16 * g, 16)
                vx[r, g16] = c_ * (_ETA * ux - p_ij * dx)
                vy[r, g16] = c_ * (_ETA * uy - p_ij * dy)
                vz[r, g16] = c_ * (_ETA * uz - p_ij * dz)
        _copy_idx(sidx, ssx, _BA)
        for r in range(_BA):
            si = ssx.at[r]
            pltpu.async_copy(vx.at[r], shx.at[si], semsc, add=True)
            pltpu.async_copy(vy.at[r], shy.at[si], semsc, add=True)
            pltpu.async_copy(vz.at[r], shz.at[si], semsc, add=True)

    _issue(shared_tab, snd_hbm, rcv_hbm, wid, 0, bufs[0], _BA)
    _issue(shared_tab, snd_hbm, rcv_hbm, wid, 1, bufs[1], _BA)

    def _outer(k2, _):
        _drain(shared_tab, bufs[0], _BA)
        _compute(0, 2 * k2)

        @pl.when(k2 < _NSTEPS // 2 - 1)
        def _():
            _issue(shared_tab, snd_hbm, rcv_hbm, wid, 2 * k2 + 2, bufs[0], _BA)

        _drain(shared_tab, bufs[1], _BA)
        _compute(1, 2 * k2 + 1)

        @pl.when(k2 < _NSTEPS // 2 - 1)
        def _():
            _issue(shared_tab, snd_hbm, rcv_hbm, wid, 2 * k2 + 3, bufs[1], _BA)
        return 0

    lax.fori_loop(0, _NSTEPS // 2, _outer, 0)
    for b in (0, 1):
        _drain_sc(b)
    plsc.subcore_barrier()

    for comp, sh in enumerate((shx, shy, shz)):
        pltpu.sync_copy(sh.at[sl], zbuf)
        pltpu.sync_copy(
            zbuf,
            out_hbm.at[pl.ds((c * _DIM + comp) * _N_PAD + s * _NSLICE, _NSLICE)])


_force_call = pl.kernel(
    _force_body,
    out_type=jax.ShapeDtypeStruct((2 * _DIM * _N_PAD,), jnp.float32),
    mesh=_mesh,
    compiler_params=_sc_params,
    scratch_types=(
        _edge_scratch(_BA) + _edge_scratch(_BA) + [
            pltpu.VMEM((_BA, _ROW), jnp.float32),
            pltpu.VMEM((_BA, _ROW), jnp.float32),
            pltpu.VMEM((_BA, _ROW), jnp.float32),
            pltpu.VMEM((_BA, _ROW), jnp.float32),
            pltpu.VMEM((_BA, _ROW), jnp.float32),
            pltpu.VMEM((_BA, _ROW), jnp.float32),
            pltpu.VMEM((_BA, _ROW), jnp.int32),
            pltpu.VMEM((_BA, _ROW), jnp.int32),
            pltpu.VMEM((_NSLICE,), jnp.float32),
            pltpu.VMEM((1600, 8), jnp.float32),
            pltpu.VMEM_SHARED((_N_PAD, 8), jnp.float32),
            pltpu.VMEM_SHARED((_N_PAD,), jnp.float32),
            pltpu.VMEM_SHARED((_N_PAD,), jnp.float32),
            pltpu.VMEM_SHARED((_N_PAD,), jnp.float32),
        ] + [pltpu.SemaphoreType.DMA] * 6
    ),
)


def _du_body(part_ref, out_ref):
    out_ref[...] = part_ref[0] + part_ref[1]


_du_call = pl.pallas_call(
    _du_body,
    grid=(_N_PAD // _BLK,),
    in_specs=[pl.BlockSpec((2, _DIM, _BLK), lambda i: (0, 0, i))],
    out_specs=pl.BlockSpec((_DIM, _BLK), lambda i: (0, i)),
    out_shape=jax.ShapeDtypeStruct((_DIM, _N_PAD), jnp.float32),
)


def kernel(abs_pos, vel_hist, senders, receivers):
    pad = jnp.full((_E_PAD - _E,), _N, jnp.int32)
    s2d = jnp.concatenate([senders, pad]).reshape(_ROWS_PAD, _ROW)
    r2d = jnp.concatenate([receivers, pad]).reshape(_ROWS_PAD, _ROW)
    pos8 = jnp.pad(abs_pos, ((0, _N_PAD - _N), (0, 5)))
    partial_rho = _rho_call(pos8, s2d, r2d).reshape(2, _N_PAD)
    table, rho, p = _pack_call(partial_rho, abs_pos, vel_hist)
    partial_du = _force_call(table, s2d, r2d)
    du3 = _du_call(partial_du.reshape(2, _DIM, _N_PAD))
    dudt = du3[:, :_N].T
    dvdt = jnp.zeros((_N, _DIM), jnp.float32)
    return (dudt, dvdt, rho[:_N], p[:_N])
